# Initial kernel scaffold; baseline (speedup 1.0000x reference)
import jax
import jax.numpy as jnp
from jax.experimental import pallas as pl


def kernel(x, edge_emb, mlp_w1, mlp_b1, ln1_g, ln1_b, mlp_w2, mlp_b2, ln2_g, ln2_b, lin_l_w, lin_l_b, lin_r_w, lin_r_b, lin_e_w, att, conv_bias, proj_w, proj_b, lnl_g, lnl_b, out_w, out_b):
    raise NotImplementedError("write your pallas kernel here")


# speedup vs baseline: 300.1595x; 300.1595x over previous
import jax
import jax.numpy as jnp
from jax import lax
from jax.experimental import pallas as pl
from jax.experimental.pallas import tpu as pltpu

_N = 128
_B = 8
_HID = 128
_HEADS = 8
_HD = 16
_L = 4
_BN = _B * _N
_JT = 64


def _ln(h, g, b):
    mu = jnp.mean(h, axis=-1, keepdims=True)
    d = h - mu
    v = jnp.mean(d * d, axis=-1, keepdims=True)
    return d * lax.rsqrt(v + 1e-5) * g + b


def _fwd(x_ref, ee_ref, w1_ref, b1_ref, g1_ref, bb1_ref, w2_ref, b2_ref,
         g2_ref, bb2_ref, llw_ref, llb_ref, lrw_ref, lrb_ref, lew_ref,
         att_ref, cb_ref, pw_ref, pb_ref, lg_ref, lb_ref, ow_ref, ob_ref,
         out_ref, agg_ref, xl_ref, xr_ref):
    f32 = jnp.float32

    def dot(a, b):
        return lax.dot(a, b, preferred_element_type=f32)

    h = _ln(dot(x_ref[:], w1_ref[:]) + b1_ref[:], g1_ref[:], bb1_ref[:])
    h = jnp.maximum(h, 0.0)
    h = _ln(dot(h, w2_ref[:]) + b2_ref[:], g2_ref[:], bb2_ref[:])

    cm_c = lax.broadcasted_iota(jnp.int32, (_HID, _HEADS), 0) // _HD
    cm_h = lax.broadcasted_iota(jnp.int32, (_HID, _HEADS), 1)
    creduce = (cm_c == cm_h).astype(f32)
    hm_h = lax.broadcasted_iota(jnp.int32, (_HEADS, _HID), 0)
    hm_c = lax.broadcasted_iota(jnp.int32, (_HEADS, _HID), 1) // _HD
    hexpand = (hm_h == hm_c).astype(f32)

    for l in range(_L):
        h_prev = h
        xl_ref[...] = dot(h, llw_ref[l]) + llb_ref[l:l + 1, :]
        xr_ref[...] = dot(h, lrw_ref[l]) + lrb_ref[l:l + 1, :]
        etab = dot(ee_ref[:], lew_ref[l])
        att_row = att_ref[l:l + 1, :]

        def b_body(b, _, etab=etab, att_row=att_row):
            xl_b = xl_ref[pl.ds(b * _N, _N), :]

            def t_body(t, _):
                j0 = t * _JT
                xr_t = xr_ref[pl.ds(b * _N + j0, _JT), :]
                jrow = lax.broadcasted_iota(jnp.int32, (_JT, 1), 0) + j0
                hi_j = jrow >= (_N // 2)
                alo_t = (xr_t + jnp.where(hi_j, etab[1:2, :], etab[0:1, :]))[:, None, :]
                ahi_t = (xr_t + jnp.where(hi_j, etab[3:4, :], etab[2:3, :]))[:, None, :]
                adg_t = (xr_t + jnp.where(hi_j, etab[5:6, :], etab[4:5, :]))[:, None, :]
                ii = lax.broadcasted_iota(jnp.int32, (_JT, _N, 1), 1)
                jj = lax.broadcasted_iota(jnp.int32, (_JT, _N, 1), 0) + j0
                a3 = jnp.where(ii < (_N // 2), alo_t, ahi_t)
                a3 = jnp.where(ii == jj, adg_t, a3)
                s = xl_b[None, :, :] + a3
                m = jnp.where(s >= 0, s, 0.2 * s)
                w = m * att_row[None, :, :]
                lg2 = dot(w.reshape(_JT * _N, _HID), creduce)
                l3 = lg2.reshape(_JT, _N, _HEADS)
                amax = jnp.max(l3, axis=1, keepdims=True)
                ex = jnp.exp(l3 - amax)
                den = jnp.sum(ex, axis=1, keepdims=True)
                alpha = ex / (den + 1e-16)
                ac = dot(alpha.reshape(_JT * _N, _HEADS), hexpand)
                ac = ac.reshape(_JT, _N, _HID)
                agg_t = jnp.sum(ac * xl_b[None, :, :], axis=1)
                agg_ref[pl.ds(b * _N + j0, _JT), :] = agg_t
                return 0

            lax.fori_loop(0, _N // _JT, t_body, 0)
            return 0

        lax.fori_loop(0, _B, b_body, 0)
        hh = agg_ref[:] + cb_ref[l:l + 1, :]
        hh = dot(hh, pw_ref[l]) + pb_ref[l:l + 1, :]
        hh = _ln(hh, lg_ref[l:l + 1, :], lb_ref[l:l + 1, :])
        hh = jnp.maximum(hh, 0.0)
        h = hh + h_prev

    hag = jnp.sum(h.reshape(_B, _N, _HID), axis=1)
    out_ref[:] = dot(hag, ow_ref[:]) + ob_ref[:]


def kernel(x, edge_emb, mlp_w1, mlp_b1, ln1_g, ln1_b, mlp_w2, mlp_b2, ln2_g,
           ln2_b, lin_l_w, lin_l_b, lin_r_w, lin_r_b, lin_e_w, att, conv_bias,
           proj_w, proj_b, lnl_g, lnl_b, out_w, out_b):
    f32 = jnp.float32
    x2 = x.reshape(_BN, 2).astype(f32)
    att_flat = att.reshape(_L, _HID)

    def row(v):
        return v.reshape(1, -1)

    return pl.pallas_call(
        _fwd,
        out_shape=jax.ShapeDtypeStruct((_B, 1), f32),
        scratch_shapes=[pltpu.VMEM((_BN, _HID), f32),
                        pltpu.VMEM((_BN, _HID), f32),
                        pltpu.VMEM((_BN, _HID), f32)],
    )(x2, edge_emb, mlp_w1, row(mlp_b1), row(ln1_g), row(ln1_b), mlp_w2,
      row(mlp_b2), row(ln2_g), row(ln2_b), lin_l_w, lin_l_b, lin_r_w,
      lin_r_b, lin_e_w, att_flat, conv_bias, proj_w, proj_b, lnl_g, lnl_b,
      out_w, out_b.reshape(1, 1))


# speedup vs baseline: 362.5580x; 1.2079x over previous
import jax
import jax.numpy as jnp
from jax import lax
from jax.experimental import pallas as pl
from jax.experimental.pallas import tpu as pltpu

_N = 128
_B = 8
_HID = 128
_HEADS = 8
_HD = 16
_L = 4
_BN = _B * _N
_JT = 64


def _ln(h, g, b):
    mu = jnp.mean(h, axis=-1, keepdims=True)
    d = h - mu
    v = jnp.mean(d * d, axis=-1, keepdims=True)
    return d * lax.rsqrt(v + 1e-5) * g + b


def _fwd(x_ref, ee_ref, w1_ref, b1_ref, g1_ref, bb1_ref, w2_ref, b2_ref,
         g2_ref, bb2_ref, llw_ref, llb_ref, lrw_ref, lrb_ref, lew_ref,
         att_ref, cb_ref, pw_ref, pb_ref, lg_ref, lb_ref, ow_ref, ob_ref,
         out_ref, agg_ref, xl_ref, xr_ref):
    f32 = jnp.float32

    def dot(a, b):
        return lax.dot(a, b, preferred_element_type=f32)

    h = _ln(dot(x_ref[:], w1_ref[:]) + b1_ref[:], g1_ref[:], bb1_ref[:])
    h = jnp.maximum(h, 0.0)
    h = _ln(dot(h, w2_ref[:]) + b2_ref[:], g2_ref[:], bb2_ref[:])

    bm_a = lax.broadcasted_iota(jnp.int32, (_HID, _HID), 0) // _HD
    bm_b = lax.broadcasted_iota(jnp.int32, (_HID, _HID), 1) // _HD
    blockmask = (bm_a == bm_b).astype(f32)

    for l in range(_L):
        h_prev = h
        xl_ref[...] = dot(h, llw_ref[l]) + llb_ref[l:l + 1, :]
        xr_ref[...] = dot(h, lrw_ref[l]) + lrb_ref[l:l + 1, :]
        etab = dot(ee_ref[:], lew_ref[l])
        cratt = blockmask * att_ref[:, l:l + 1]

        def b_body(b, _, etab=etab, cratt=cratt):
            xl_b = xl_ref[pl.ds(b * _N, _N), :]

            def t_body(t, _):
                j0 = t * _JT
                xr_t = xr_ref[pl.ds(b * _N + j0, _JT), :]
                jrow = lax.broadcasted_iota(jnp.int32, (_JT, 1), 0) + j0
                hi_j = jrow >= (_N // 2)
                alo_t = (xr_t + jnp.where(hi_j, etab[1:2, :], etab[0:1, :]))[:, None, :]
                ahi_t = (xr_t + jnp.where(hi_j, etab[3:4, :], etab[2:3, :]))[:, None, :]
                adg_t = (xr_t + jnp.where(hi_j, etab[5:6, :], etab[4:5, :]))[:, None, :]
                ii = lax.broadcasted_iota(jnp.int32, (_JT, _N, 1), 1)
                jj = lax.broadcasted_iota(jnp.int32, (_JT, _N, 1), 0) + j0
                a3 = jnp.where(ii < (_N // 2), alo_t, ahi_t)
                a3 = jnp.where(ii == jj, adg_t, a3)
                s = xl_b[None, :, :] + a3
                m = jnp.where(s >= 0, s, 0.2 * s)
                lc = dot(m.reshape(_JT * _N, _HID), cratt)
                lc = lc.reshape(_JT, _N, _HID)
                amax = jnp.max(lc, axis=1, keepdims=True)
                ex = jnp.exp(lc - amax)
                den = jnp.sum(ex, axis=1, keepdims=True)
                alpha = ex / (den + 1e-16)
                agg_t = jnp.sum(alpha * xl_b[None, :, :], axis=1)
                agg_ref[pl.ds(b * _N + j0, _JT), :] = agg_t
                return 0

            lax.fori_loop(0, _N // _JT, t_body, 0)
            return 0

        lax.fori_loop(0, _B, b_body, 0)
        hh = agg_ref[:] + cb_ref[l:l + 1, :]
        hh = dot(hh, pw_ref[l]) + pb_ref[l:l + 1, :]
        hh = _ln(hh, lg_ref[l:l + 1, :], lb_ref[l:l + 1, :])
        hh = jnp.maximum(hh, 0.0)
        h = hh + h_prev

    hag = jnp.sum(h.reshape(_B, _N, _HID), axis=1)
    out_ref[:] = dot(hag, ow_ref[:]) + ob_ref[:]


def kernel(x, edge_emb, mlp_w1, mlp_b1, ln1_g, ln1_b, mlp_w2, mlp_b2, ln2_g,
           ln2_b, lin_l_w, lin_l_b, lin_r_w, lin_r_b, lin_e_w, att, conv_bias,
           proj_w, proj_b, lnl_g, lnl_b, out_w, out_b):
    f32 = jnp.float32
    x2 = x.reshape(_BN, 2).astype(f32)
    att_t = att.reshape(_L, _HID).T

    def row(v):
        return v.reshape(1, -1)

    return pl.pallas_call(
        _fwd,
        out_shape=jax.ShapeDtypeStruct((_B, 1), f32),
        scratch_shapes=[pltpu.VMEM((_BN, _HID), f32),
                        pltpu.VMEM((_BN, _HID), f32),
                        pltpu.VMEM((_BN, _HID), f32)],
    )(x2, edge_emb, mlp_w1, row(mlp_b1), row(ln1_g), row(ln1_b), mlp_w2,
      row(mlp_b2), row(ln2_g), row(ln2_b), lin_l_w, lin_l_b, lin_r_w,
      lin_r_b, lin_e_w, att_t, conv_bias, proj_w, proj_b, lnl_g, lnl_b,
      out_w, out_b.reshape(1, 1))


# speedup vs baseline: 370.8830x; 1.0230x over previous
import jax
import jax.numpy as jnp
from jax import lax
from jax.experimental import pallas as pl
from jax.experimental.pallas import tpu as pltpu

_N = 128
_B = 8
_HID = 128
_HEADS = 8
_HD = 16
_L = 4
_BN = _B * _N
_JT = 64


def _ln(h, g, b):
    mu = jnp.mean(h, axis=-1, keepdims=True)
    d = h - mu
    v = jnp.mean(d * d, axis=-1, keepdims=True)
    return d * lax.rsqrt(v + 1e-5) * g + b


def _fwd(x_ref, ee_ref, w1_ref, b1_ref, g1_ref, bb1_ref, w2_ref, b2_ref,
         g2_ref, bb2_ref, llw_ref, llb_ref, lrw_ref, lrb_ref, lew_ref,
         att_ref, cb_ref, pw_ref, pb_ref, lg_ref, lb_ref, ow_ref, ob_ref,
         out_ref, agg_ref, xl_ref, xr_ref):
    f32 = jnp.float32

    def dot(a, b):
        return lax.dot(a, b, preferred_element_type=f32)

    h = _ln(dot(x_ref[:], w1_ref[:]) + b1_ref[:], g1_ref[:], bb1_ref[:])
    h = jnp.maximum(h, 0.0)
    h = _ln(dot(h, w2_ref[:]) + b2_ref[:], g2_ref[:], bb2_ref[:])

    bm_a = lax.broadcasted_iota(jnp.int32, (_HID, _HID), 0) // _HD
    bm_b = lax.broadcasted_iota(jnp.int32, (_HID, _HID), 1) // _HD
    blockmask = (bm_a == bm_b).astype(f32)

    for l in range(_L):
        h_prev = h
        xl_ref[...] = dot(h, llw_ref[l]) + llb_ref[l:l + 1, :]
        xr_ref[...] = dot(h, lrw_ref[l]) + lrb_ref[l:l + 1, :]
        etab = dot(ee_ref[:], lew_ref[l])
        cratt = blockmask * att_ref[:, l:l + 1]

        def b_body(b, _, etab=etab, cratt=cratt):
            xl_b = xl_ref[pl.ds(b * _N, _N), :]
            irow = lax.broadcasted_iota(jnp.int32, (_N, 1), 0)
            lo_i = irow < (_N // 2)
            ii = lax.broadcasted_iota(jnp.int32, (_JT, _N, 1), 1)
            jj = lax.broadcasted_iota(jnp.int32, (_JT, _N, 1), 0)

            for t in range(_N // _JT):
                j0 = t * _JT
                xr_t = xr_ref[pl.ds(b * _N + j0, _JT), :]
                xlE = jnp.where(lo_i, xl_b + etab[t:t + 1, :],
                                xl_b + etab[2 + t:3 + t, :])
                s = xr_t[:, None, :] + xlE[None, :, :]
                m = jnp.maximum(s, 0.2 * s)
                lc = dot(m.reshape(_JT * _N, _HID), cratt)
                lc = lc.reshape(_JT, _N, _HID)
                xl_t = xl_ref[pl.ds(b * _N + j0, _JT), :]
                s_d = xl_t + xr_t + etab[4 + t:5 + t, :]
                m_d = jnp.maximum(s_d, 0.2 * s_d)
                lcd = dot(m_d, cratt)
                lc = jnp.where(ii == jj + j0, lcd[:, None, :], lc)
                amax = jnp.max(lc, axis=1, keepdims=True)
                ex = jnp.exp(lc - amax)
                den = jnp.sum(ex, axis=1, keepdims=True)
                rec = 1.0 / (den + 1e-16)
                agg_t = jnp.sum((ex * rec) * xl_b[None, :, :], axis=1)
                agg_ref[pl.ds(b * _N + j0, _JT), :] = agg_t
            return 0

        lax.fori_loop(0, _B, b_body, 0)
        hh = agg_ref[:] + cb_ref[l:l + 1, :]
        hh = dot(hh, pw_ref[l]) + pb_ref[l:l + 1, :]
        hh = _ln(hh, lg_ref[l:l + 1, :], lb_ref[l:l + 1, :])
        hh = jnp.maximum(hh, 0.0)
        h = hh + h_prev

    hag = jnp.sum(h.reshape(_B, _N, _HID), axis=1)
    out_ref[:] = dot(hag, ow_ref[:]) + ob_ref[:]


def kernel(x, edge_emb, mlp_w1, mlp_b1, ln1_g, ln1_b, mlp_w2, mlp_b2, ln2_g,
           ln2_b, lin_l_w, lin_l_b, lin_r_w, lin_r_b, lin_e_w, att, conv_bias,
           proj_w, proj_b, lnl_g, lnl_b, out_w, out_b):
    f32 = jnp.float32
    x2 = x.reshape(_BN, 2).astype(f32)
    att_t = att.reshape(_L, _HID).T

    def row(v):
        return v.reshape(1, -1)

    return pl.pallas_call(
        _fwd,
        out_shape=jax.ShapeDtypeStruct((_B, 1), f32),
        scratch_shapes=[pltpu.VMEM((_BN, _HID), f32),
                        pltpu.VMEM((_BN, _HID), f32),
                        pltpu.VMEM((_BN, _HID), f32)],
    )(x2, edge_emb, mlp_w1, row(mlp_b1), row(ln1_g), row(ln1_b), mlp_w2,
      row(mlp_b2), row(ln2_g), row(ln2_b), lin_l_w, lin_l_b, lin_r_w,
      lin_r_b, lin_e_w, att_t, conv_bias, proj_w, proj_b, lnl_g, lnl_b,
      out_w, out_b.reshape(1, 1))
